# Initial kernel scaffold; baseline (speedup 1.0000x reference)
import jax
import jax.numpy as jnp
from jax.experimental import pallas as pl


def kernel(x, edge_index, edge_type, W, b):
    raise NotImplementedError("write your pallas kernel here")


# speedup vs baseline: 8.0041x; 8.0041x over previous
import functools

import jax
import jax.numpy as jnp
from jax import lax
from jax.experimental import pallas as pl
from jax.experimental.pallas import tpu as pltpu
from jax.experimental.pallas import tpu_sc as plsc

N = 10000
NP = 10240
E = 320000
D = 128
DH = D // 2
R = 8

NC, NS, L = 2, 16, 16
NW = NC * NS
EPT = E // NW
CB = 80
NB = EPT // CB
ROWS_PT = NP // NS
DEG_SLICE = (R * NP) // NS

_mesh = plsc.VectorSubcoreMesh(core_axis_name="c", subcore_axis_name="s")
_sc_params = pltpu.CompilerParams(needs_layout_passes=False,
                                  use_tc_tiling_on_sc=False)


@functools.partial(
    pl.kernel,
    out_type=[
        jax.ShapeDtypeStruct((NC * R * NP,), jnp.float32),
        jax.ShapeDtypeStruct((NC * R * NP,), jnp.float32),
    ],
    mesh=_mesh,
    compiler_params=_sc_params,
    scratch_types=[
        pltpu.VMEM((CB,), jnp.int32),
        pltpu.VMEM((CB,), jnp.int32),
        pltpu.VMEM((CB,), jnp.int32),
        pltpu.VMEM((CB,), jnp.int32),
        pltpu.VMEM((CB,), jnp.int32),
        pltpu.VMEM((CB,), jnp.float32),
        pltpu.VMEM((DEG_SLICE,), jnp.float32),
        pltpu.VMEM_SHARED((R * NP,), jnp.float32),
        pltpu.VMEM_SHARED((R * NP,), jnp.float32),
    ],
)
def _deg_call(edges, deg_o_out, deg_i_out, sb, db, tb, gidx, didx, ones,
              zbuf, deg_o_sh, deg_i_sh):
    cid = lax.axis_index("c")
    sid = lax.axis_index("s")
    wid = sid * NC + cid

    ones16 = jnp.ones((L,), jnp.float32)
    zeros16 = jnp.zeros((L,), jnp.float32)

    def _fill(i, _):
        ones[pl.ds(i * L, L)] = ones16
        return 0
    lax.fori_loop(0, CB // L, _fill, 0)

    def _zfill(i, _):
        zbuf[pl.ds(i * L, L)] = zeros16
        return 0
    lax.fori_loop(0, DEG_SLICE // L, _zfill, 0)

    my_deg = pl.ds(sid * DEG_SLICE, DEG_SLICE)
    pltpu.sync_copy(zbuf, deg_o_sh.at[my_deg])
    pltpu.sync_copy(zbuf, deg_i_sh.at[my_deg])
    plsc.subcore_barrier()

    base_e = wid * EPT

    def _blk(bi, _):
        off = base_e + bi * CB
        pltpu.sync_copy(edges.at[pl.ds(off, CB)], sb)
        pltpu.sync_copy(edges.at[pl.ds(E + off, CB)], db)
        pltpu.sync_copy(edges.at[pl.ds(2 * E + off, CB)], tb)

        def _grp(g, _):
            s16 = sb[pl.ds(g * L, L)]
            d16 = db[pl.ds(g * L, L)]
            t16 = tb[pl.ds(g * L, L)]
            gidx[pl.ds(g * L, L)] = t16 * NP + s16
            didx[pl.ds(g * L, L)] = t16 * NP + d16
            return 0
        lax.fori_loop(0, CB // L, _grp, 0)

        pltpu.sync_copy(ones, deg_o_sh.at[gidx], add=True)
        pltpu.sync_copy(ones, deg_i_sh.at[didx], add=True)
        return 0
    lax.fori_loop(0, NB, _blk, 0)

    plsc.subcore_barrier()
    out_sl = pl.ds(cid * (R * NP) + sid * DEG_SLICE, DEG_SLICE)
    pltpu.sync_copy(deg_o_sh.at[my_deg], deg_o_out.at[out_sl])
    pltpu.sync_copy(deg_i_sh.at[my_deg], deg_i_out.at[out_sl])


BN = 2048


def _mm_body(deg_o_ref, deg_i_ref, x_ref, w_ref, h_ref, ndv_ref):
    deg_o = deg_o_ref[0, 0] + deg_o_ref[1, 0]
    deg_i = deg_i_ref[0, 0] + deg_i_ref[1, 0]
    ns = lax.rsqrt(jnp.clip(deg_o, 1.0, None))
    ndv_ref[0] = lax.rsqrt(jnp.clip(deg_i, 1.0, None))
    xs = x_ref[...] * ns
    h_ref[0] = jnp.dot(xs, w_ref[0],
                       preferred_element_type=jnp.float32,
                       precision=lax.Precision.HIGHEST)


_mm_call = pl.pallas_call(
    _mm_body,
    grid=(R, NP // BN),
    in_specs=[
        pl.BlockSpec((NC, 1, BN, 1), lambda r, i: (0, r, i, 0)),
        pl.BlockSpec((NC, 1, BN, 1), lambda r, i: (0, r, i, 0)),
        pl.BlockSpec((BN, D), lambda r, i: (i, 0)),
        pl.BlockSpec((1, D, D), lambda r, i: (r, 0, 0)),
    ],
    out_specs=[
        pl.BlockSpec((1, BN, D), lambda r, i: (r, i, 0)),
        pl.BlockSpec((1, BN, 1), lambda r, i: (r, i, 0)),
    ],
    out_shape=[
        jax.ShapeDtypeStruct((R, NP, D), jnp.float32),
        jax.ShapeDtypeStruct((R, NP, 1), jnp.float32),
    ],
)


@functools.partial(
    pl.kernel,
    out_type=jax.ShapeDtypeStruct((2, NC, NP, DH), jnp.float32),
    mesh=_mesh,
    compiler_params=_sc_params,
    scratch_types=[
        pltpu.VMEM((R * NP,), jnp.float32),
        pltpu.VMEM((CB,), jnp.int32),
        pltpu.VMEM((CB,), jnp.int32),
        pltpu.VMEM((CB,), jnp.int32),
        pltpu.VMEM((CB,), jnp.int32),
        pltpu.VMEM((CB,), jnp.int32),
        pltpu.VMEM((CB,), jnp.float32),
        pltpu.VMEM((CB, DH), jnp.float32),
        pltpu.VMEM_SHARED((NP, DH), jnp.float32),
        pltpu.SemaphoreType.DMA,
    ],
)
def _agg_call(edges, hf, ndv, part_out, ndv_v, sb, db, tb, gidx, didx, wv,
              rows, acc_sh, sem):
    cid = lax.axis_index("c")
    sid = lax.axis_index("s")
    wid = sid * NC + cid

    pltpu.sync_copy(ndv, ndv_v)

    zeros16 = jnp.zeros((L,), jnp.float32)
    base_r = sid * ROWS_PT
    base_e = wid * EPT

    for half in range(2):
        def _zrow(i, _):
            for k in range(DH // L):
                rows[i, pl.ds(k * L, L)] = zeros16
            return 0
        lax.fori_loop(0, CB, _zrow, 0)
        for j in range(ROWS_PT // CB):
            pltpu.sync_copy(rows, acc_sh.at[pl.ds(base_r + j * CB, CB)])
        plsc.subcore_barrier()

        def _blk(bi, _):
            off = base_e + bi * CB
            pltpu.sync_copy(edges.at[pl.ds(off, CB)], sb)
            pltpu.sync_copy(edges.at[pl.ds(E + off, CB)], db)
            pltpu.sync_copy(edges.at[pl.ds(2 * E + off, CB)], tb)

            def _grp(g, _):
                s16 = sb[pl.ds(g * L, L)]
                d16 = db[pl.ds(g * L, L)]
                t16 = tb[pl.ds(g * L, L)]
                gidx[pl.ds(g * L, L)] = (t16 * NP + s16) * 2 + half
                didx[pl.ds(g * L, L)] = d16
                wv[pl.ds(g * L, L)] = plsc.load_gather(ndv_v, [t16 * NP + d16])
                return 0
            lax.fori_loop(0, CB // L, _grp, 0)

            pltpu.async_copy(hf.at[gidx], rows, sem).wait()

            def _scale(g, _):
                for j in range(L):
                    jj = g * L + j
                    wj = plsc.load_gather(wv, [jnp.zeros((L,), jnp.int32) + jj])
                    for k in range(DH // L):
                        rows[jj, pl.ds(k * L, L)] = (
                            rows[jj, pl.ds(k * L, L)] * wj)
                return 0
            lax.fori_loop(0, CB // L, _scale, 0)

            pltpu.sync_copy(rows, acc_sh.at[didx], add=True)
            return 0
        lax.fori_loop(0, NB, _blk, 0)

        plsc.subcore_barrier()
        for j in range(ROWS_PT // CB):
            sl = pl.ds(base_r + j * CB, CB)
            pltpu.sync_copy(acc_sh.at[sl], part_out.at[half, cid, sl])


BD = 2000


def _fin_body(p_ref, b_ref, o_ref):
    bias = jnp.sum(b_ref[...], axis=0, keepdims=True)
    left = p_ref[0, 0] + p_ref[0, 1]
    right = p_ref[1, 0] + p_ref[1, 1]
    o_ref[...] = jnp.concatenate([left, right], axis=1) + bias


_fin_call = pl.pallas_call(
    _fin_body,
    grid=(N // BD,),
    in_specs=[
        pl.BlockSpec((2, NC, BD, DH), lambda i: (0, 0, i, 0)),
        pl.BlockSpec((R, D), lambda i: (0, 0)),
    ],
    out_specs=pl.BlockSpec((BD, D), lambda i: (i, 0)),
    out_shape=jax.ShapeDtypeStruct((N, D), jnp.float32),
)


def kernel(x, edge_index, edge_type, W, b):
    edges = jnp.concatenate(
        [edge_index[0], edge_index[1], edge_type]).astype(jnp.int32)
    x_pad = jnp.pad(x, ((0, NP - N), (0, 0)))
    deg_o, deg_i = _deg_call(edges)
    h, ndv = _mm_call(deg_o.reshape(NC, R, NP, 1),
                      deg_i.reshape(NC, R, NP, 1), x_pad, W)
    part = _agg_call(edges, h.reshape(R * NP * 2, DH), ndv.reshape(R * NP))
    return _fin_call(part, b)


# speedup vs baseline: 13.2511x; 1.6555x over previous
import functools

import jax
import jax.numpy as jnp
from jax import lax
from jax.experimental import pallas as pl
from jax.experimental.pallas import tpu as pltpu
from jax.experimental.pallas import tpu_sc as plsc

N = 10000
NP = 10240
E = 320000
D = 128
DH = D // 2
R = 8

NC, NS, L = 2, 16, 16
NW = NC * NS
EPT = E // NW
CB = 80
NB = EPT // CB
ROWS_PT = NP // NS
DEG_SLICE = (R * NP) // NS

_mesh = plsc.VectorSubcoreMesh(core_axis_name="c", subcore_axis_name="s")
_sc_params = pltpu.CompilerParams(needs_layout_passes=False,
                                  use_tc_tiling_on_sc=False)


@functools.partial(
    pl.kernel,
    out_type=[
        jax.ShapeDtypeStruct((NC * R * NP,), jnp.float32),
        jax.ShapeDtypeStruct((NC * R * NP,), jnp.float32),
    ],
    mesh=_mesh,
    compiler_params=_sc_params,
    scratch_types=[
        pltpu.VMEM((3 * CB,), jnp.int32),
        pltpu.VMEM((CB,), jnp.int32),
        pltpu.VMEM((CB,), jnp.int32),
        pltpu.VMEM((CB,), jnp.float32),
        pltpu.VMEM((DEG_SLICE,), jnp.float32),
        pltpu.VMEM_SHARED((R * NP,), jnp.float32),
        pltpu.VMEM_SHARED((R * NP,), jnp.float32),
    ],
)
def _deg_call(edges, deg_o_out, deg_i_out, eb, gidx, didx, ones,
              zbuf, deg_o_sh, deg_i_sh):
    cid = lax.axis_index("c")
    sid = lax.axis_index("s")
    wid = sid * NC + cid

    ones16 = jnp.ones((L,), jnp.float32)
    zeros16 = jnp.zeros((L,), jnp.float32)

    def _fill(i, _):
        ones[pl.ds(i * L, L)] = ones16
        return 0
    lax.fori_loop(0, CB // L, _fill, 0)

    def _zfill(i, _):
        zbuf[pl.ds(i * L, L)] = zeros16
        return 0
    lax.fori_loop(0, DEG_SLICE // L, _zfill, 0)

    my_deg = pl.ds(sid * DEG_SLICE, DEG_SLICE)
    pltpu.sync_copy(zbuf, deg_o_sh.at[my_deg])
    pltpu.sync_copy(zbuf, deg_i_sh.at[my_deg])
    plsc.subcore_barrier()

    base_b = wid * NB

    def _blk(bi, _):
        pltpu.sync_copy(edges.at[pl.ds((base_b + bi) * 3 * CB, 3 * CB)], eb)

        def _grp(g, _):
            s16 = eb[pl.ds(g * L, L)]
            d16 = eb[pl.ds(CB + g * L, L)]
            t16 = eb[pl.ds(2 * CB + g * L, L)]
            gidx[pl.ds(g * L, L)] = t16 * NP + s16
            didx[pl.ds(g * L, L)] = t16 * NP + d16
            return 0
        lax.fori_loop(0, CB // L, _grp, 0)

        pltpu.sync_copy(ones, deg_o_sh.at[gidx], add=True)
        pltpu.sync_copy(ones, deg_i_sh.at[didx], add=True)
        return 0
    lax.fori_loop(0, NB, _blk, 0)

    plsc.subcore_barrier()
    out_sl = pl.ds(cid * (R * NP) + sid * DEG_SLICE, DEG_SLICE)
    pltpu.sync_copy(deg_o_sh.at[my_deg], deg_o_out.at[out_sl])
    pltpu.sync_copy(deg_i_sh.at[my_deg], deg_i_out.at[out_sl])


BN = 2048


def _mm_body(deg_o_ref, deg_i_ref, x_ref, w_ref, h_ref, ndv_ref):
    deg_o = deg_o_ref[0, 0] + deg_o_ref[1, 0]
    deg_i = deg_i_ref[0, 0] + deg_i_ref[1, 0]
    ns = lax.rsqrt(jnp.clip(deg_o, 1.0, None))
    ndv_ref[0] = lax.rsqrt(jnp.clip(deg_i, 1.0, None))
    xs = x_ref[...] * ns
    h_ref[0] = jnp.dot(xs, w_ref[0],
                       preferred_element_type=jnp.float32,
                       precision=lax.Precision.HIGHEST)


_mm_call = pl.pallas_call(
    _mm_body,
    grid=(R, NP // BN),
    in_specs=[
        pl.BlockSpec((NC, 1, BN, 1), lambda r, i: (0, r, i, 0)),
        pl.BlockSpec((NC, 1, BN, 1), lambda r, i: (0, r, i, 0)),
        pl.BlockSpec((BN, D), lambda r, i: (i, 0)),
        pl.BlockSpec((1, D, D), lambda r, i: (r, 0, 0)),
    ],
    out_specs=[
        pl.BlockSpec((1, BN, D), lambda r, i: (r, i, 0)),
        pl.BlockSpec((1, BN, 1), lambda r, i: (r, i, 0)),
    ],
    out_shape=[
        jax.ShapeDtypeStruct((R, NP, D), jnp.float32),
        jax.ShapeDtypeStruct((R, NP, 1), jnp.float32),
    ],
)


@functools.partial(
    pl.kernel,
    out_type=jax.ShapeDtypeStruct((2, NC, NP, DH), jnp.float32),
    mesh=_mesh,
    compiler_params=_sc_params,
    scratch_types=[
        pltpu.VMEM((2, 3 * CB), jnp.int32),
        pltpu.VMEM((2, CB), jnp.int32),
        pltpu.VMEM((2, CB), jnp.int32),
        pltpu.VMEM((2, CB), jnp.int32),
        pltpu.VMEM((2, CB), jnp.float32),
        pltpu.VMEM((2 * CB, DH), jnp.float32),
        pltpu.VMEM_SHARED((NP, DH), jnp.float32),
        pltpu.SemaphoreType.DMA((2,)),
        pltpu.SemaphoreType.DMA((2,)),
        pltpu.SemaphoreType.DMA((2,)),
        pltpu.SemaphoreType.DMA((2,)),
    ],
)
def _agg_call(edges, hf, ndv, part_out, eb, gidx, didx, widx, wv,
              rows, acc_sh, esem, gsem, wsem, ssem):
    cid = lax.axis_index("c")
    sid = lax.axis_index("s")
    wid = sid * NC + cid

    zeros16 = jnp.zeros((L,), jnp.float32)
    base_r = sid * ROWS_PT
    base_b = wid * NB

    def _stage(bi, buf):
        pltpu.async_copy(edges.at[pl.ds((base_b + bi) * 3 * CB, 3 * CB)],
                         eb.at[buf], esem.at[buf])

    def _wait_stage(buf):
        pltpu.make_async_copy(edges.at[pl.ds(0, 3 * CB)], eb.at[buf],
                              esem.at[buf]).wait()

    def _wait_rowdma(sem, buf):
        pltpu.make_async_copy(hf.at[pl.ds(0, CB)],
                              rows.at[pl.ds(buf * CB, CB)],
                              sem.at[buf]).wait()

    for half in range(2):
        def _zrow(i, _):
            for k in range(DH // L):
                rows[i, pl.ds(k * L, L)] = zeros16
            return 0
        lax.fori_loop(0, CB, _zrow, 0)
        for j in range(ROWS_PT // CB):
            pltpu.sync_copy(rows.at[pl.ds(0, CB)],
                            acc_sh.at[pl.ds(base_r + j * CB, CB)])
        plsc.subcore_barrier()

        def _grp_block(buf):
            def _grp(g, _):
                s16 = eb[buf, pl.ds(g * L, L)]
                d16 = eb[buf, pl.ds(CB + g * L, L)]
                t16 = eb[buf, pl.ds(2 * CB + g * L, L)]
                gidx[buf, pl.ds(g * L, L)] = (t16 * NP + s16) * 2 + half
                didx[buf, pl.ds(g * L, L)] = d16
                widx[buf, pl.ds(g * L, L)] = t16 * NP + d16
                return 0
            lax.fori_loop(0, CB // L, _grp, 0)

        def _start_gathers(buf):
            pltpu.async_copy(hf.at[gidx.at[buf]],
                             rows.at[pl.ds(buf * CB, CB)], gsem.at[buf])
            pltpu.async_copy(ndv.at[widx.at[buf]], wv.at[buf], wsem.at[buf])

        _stage(0, 0)
        _wait_stage(0)
        _grp_block(0)
        _start_gathers(0)
        _stage(1, 1)

        def _blk(bi, _):
            b = bi & 1
            nb = 1 - b

            @pl.when(bi + 1 < NB)
            def _():
                _wait_stage(nb)
                _grp_block(nb)

                @pl.when(bi >= 1)
                def _():
                    _wait_rowdma(ssem, nb)
                _start_gathers(nb)

                @pl.when(bi + 2 < NB)
                def _():
                    _stage(bi + 2, b)

            _wait_rowdma(gsem, b)
            pltpu.make_async_copy(ndv.at[pl.ds(0, CB)], wv.at[b],
                                  wsem.at[b]).wait()
            def _scale(g, _):
                for j in range(L):
                    jj = g * L + j
                    wj = plsc.load_gather(
                        wv, [jnp.zeros((L,), jnp.int32) + b,
                             jnp.zeros((L,), jnp.int32) + jj])
                    for k in range(DH // L):
                        row = b * CB + jj
                        rows[row, pl.ds(k * L, L)] = (
                            rows[row, pl.ds(k * L, L)] * wj)
                return 0
            lax.fori_loop(0, CB // L, _scale, 0)

            pltpu.async_copy(rows.at[pl.ds(b * CB, CB)],
                             acc_sh.at[didx.at[b]], ssem.at[b], add=True)
            return 0
        lax.fori_loop(0, NB, _blk, 0)

        _wait_rowdma(ssem, 1)
        _wait_rowdma(ssem, 0)

        plsc.subcore_barrier()
        for j in range(ROWS_PT // CB):
            sl = pl.ds(base_r + j * CB, CB)
            pltpu.sync_copy(acc_sh.at[sl], part_out.at[half, cid, sl])


BD = 2000


def _fin_body(p_ref, b_ref, o_ref):
    bias = jnp.sum(b_ref[...], axis=0, keepdims=True)
    left = p_ref[0, 0] + p_ref[0, 1]
    right = p_ref[1, 0] + p_ref[1, 1]
    o_ref[...] = jnp.concatenate([left, right], axis=1) + bias


_fin_call = pl.pallas_call(
    _fin_body,
    grid=(N // BD,),
    in_specs=[
        pl.BlockSpec((2, NC, BD, DH), lambda i: (0, 0, i, 0)),
        pl.BlockSpec((R, D), lambda i: (0, 0)),
    ],
    out_specs=pl.BlockSpec((BD, D), lambda i: (i, 0)),
    out_shape=jax.ShapeDtypeStruct((N, D), jnp.float32),
)


def kernel(x, edge_index, edge_type, W, b):
    edges = (jnp.concatenate([edge_index[0], edge_index[1], edge_type])
             .astype(jnp.int32)
             .reshape(3, E // CB, CB)
             .transpose(1, 0, 2)
             .reshape(-1))
    x_pad = jnp.pad(x, ((0, NP - N), (0, 0)))
    deg_o, deg_i = _deg_call(edges)
    h, ndv = _mm_call(deg_o.reshape(NC, R, NP, 1),
                      deg_i.reshape(NC, R, NP, 1), x_pad, W)
    part = _agg_call(edges, h.reshape(R * NP * 2, DH), ndv.reshape(R * NP))
    return _fin_call(part, b)
